# Initial kernel scaffold; baseline (speedup 1.0000x reference)
import functools

import jax
import jax.numpy as jnp
from jax.experimental import pallas as pl
from jax.experimental.pallas import tpu as pltpu


def _moe_dense_body(x_ref, gw_ref, gu_ref, dn_ref, out_ref):
    e = pl.program_id(1)

    x = x_ref[...]

    logits = jnp.dot(x, gw_ref[...], preferred_element_type=jnp.float32)
    p = jax.nn.softmax(logits, axis=-1)
    m1 = jnp.max(p, axis=-1, keepdims=True)
    eq1 = p == m1
    first1 = eq1 & (jnp.cumsum(eq1.astype(jnp.int32), axis=-1) == 1)
    p2 = jnp.where(first1, -jnp.inf, p)
    m2 = jnp.max(p2, axis=-1, keepdims=True)
    eq2 = p2 == m2
    first2 = eq2 & (jnp.cumsum(eq2.astype(jnp.int32), axis=-1) == 1)
    sel = first1 | first2
    w_all = jnp.where(sel, p, 0.0) / (m1 + m2)
    w_col = jax.lax.dynamic_slice_in_dim(w_all, e, 1, axis=1)

    xb = x.astype(jnp.bfloat16)
    gu = jnp.dot(xb, gu_ref[0], preferred_element_type=jnp.float32)
    inter = gu.shape[-1] // 2
    gate = gu[:, :inter]
    up = gu[:, inter:]
    h = (gate * jax.lax.logistic(gate)) * up
    o = jnp.dot(h.astype(jnp.bfloat16), dn_ref[0],
                preferred_element_type=jnp.float32)
    contrib = o * w_col

    @pl.when(e == 0)
    def _():
        out_ref[...] = contrib

    @pl.when(e != 0)
    def _():
        out_ref[...] += contrib


@jax.jit
def kernel(hidden_states, gate_up_proj, down_proj, gate_weight):
    B, S, H = hidden_states.shape
    E, _, I2 = gate_up_proj.shape
    T = B * S
    x = hidden_states.reshape(T, H)

    tile_t = min(1024, T)
    assert T % tile_t == 0
    grid = (T // tile_t, E)

    gu_b = gate_up_proj.astype(jnp.bfloat16)
    dn_b = down_proj.astype(jnp.bfloat16)

    out = pl.pallas_call(
        _moe_dense_body,
        grid=grid,
        in_specs=[
            pl.BlockSpec((tile_t, H), lambda t, e: (t, 0)),
            pl.BlockSpec((H, E), lambda t, e: (0, 0)),
            pl.BlockSpec((1, H, I2), lambda t, e: (e, 0, 0)),
            pl.BlockSpec((1, I2 // 2, H), lambda t, e: (e, 0, 0)),
        ],
        out_specs=pl.BlockSpec((tile_t, H), lambda t, e: (t, 0)),
        out_shape=jax.ShapeDtypeStruct((T, H), jnp.float32),
        compiler_params=pltpu.CompilerParams(
            dimension_semantics=("arbitrary", "arbitrary"),
        ),
    )(x, gate_weight, gu_b, dn_b)

    return out.reshape(B, S, H)


# speedup vs baseline: 1.0616x; 1.0616x over previous
import functools

import jax
import jax.numpy as jnp
from jax.experimental import pallas as pl
from jax.experimental.pallas import tpu as pltpu


def _moe_dense_body(x_ref, gw_ref, gu_ref, dn_ref, out_ref):
    e = pl.program_id(1)

    x = x_ref[...]

    logits = jnp.dot(x, gw_ref[...], preferred_element_type=jnp.float32)
    p = jax.nn.softmax(logits, axis=-1)
    n_e = p.shape[-1]
    iota = jax.lax.broadcasted_iota(jnp.int32, p.shape, dimension=1)
    m1 = jnp.max(p, axis=-1, keepdims=True)
    eq1 = p == m1
    idx1 = jnp.min(jnp.where(eq1, iota, n_e), axis=-1, keepdims=True)
    first1 = iota == idx1
    p2 = jnp.where(first1, -jnp.inf, p)
    m2 = jnp.max(p2, axis=-1, keepdims=True)
    eq2 = p2 == m2
    idx2 = jnp.min(jnp.where(eq2, iota, n_e), axis=-1, keepdims=True)
    first2 = iota == idx2
    sel = first1 | first2
    w_all = jnp.where(sel, p, 0.0) / (m1 + m2)
    w_col = jnp.sum(jnp.where(iota == e, w_all, 0.0), axis=-1,
                    keepdims=True)

    xb = x.astype(jnp.bfloat16)
    gu = jnp.dot(xb, gu_ref[0], preferred_element_type=jnp.float32)
    inter = gu.shape[-1] // 2
    gate = gu[:, :inter]
    up = gu[:, inter:]
    h = (gate * jax.lax.logistic(gate)) * up
    o = jnp.dot(h.astype(jnp.bfloat16), dn_ref[0],
                preferred_element_type=jnp.float32)
    contrib = o * w_col

    @pl.when(e == 0)
    def _():
        out_ref[...] = contrib

    @pl.when(e != 0)
    def _():
        out_ref[...] += contrib


@jax.jit
def kernel(hidden_states, gate_up_proj, down_proj, gate_weight):
    B, S, H = hidden_states.shape
    E, _, I2 = gate_up_proj.shape
    T = B * S
    x = hidden_states.reshape(T, H)

    tile_t = min(512, T)
    assert T % tile_t == 0
    grid = (T // tile_t, E)

    gu_b = gate_up_proj.astype(jnp.bfloat16)
    dn_b = down_proj.astype(jnp.bfloat16)

    out = pl.pallas_call(
        _moe_dense_body,
        grid=grid,
        in_specs=[
            pl.BlockSpec((tile_t, H), lambda t, e: (t, 0)),
            pl.BlockSpec((H, E), lambda t, e: (0, 0)),
            pl.BlockSpec((1, H, I2), lambda t, e: (e, 0, 0)),
            pl.BlockSpec((1, I2 // 2, H), lambda t, e: (e, 0, 0)),
        ],
        out_specs=pl.BlockSpec((tile_t, H), lambda t, e: (t, 0)),
        out_shape=jax.ShapeDtypeStruct((T, H), jnp.float32),
        compiler_params=pltpu.CompilerParams(
            dimension_semantics=("arbitrary", "arbitrary"),
        ),
    )(x, gate_weight, gu_b, dn_b)

    return out.reshape(B, S, H)
